# baseline (speedup 1.0000x reference)
import functools

import jax
import jax.numpy as jnp
from jax import lax
from jax.experimental import pallas as pl
from jax.experimental.pallas import tpu as pltpu
from jax.experimental.pallas import tpu_sc as plsc

B = 4096
D = 64
HID = 128
POSTER_DIM = 2048
NROW = 100000
PACK = 4
DPK = 128
NPK = NROW // PACK

_NC, _NS = 2, 16
NW = _NC * _NS
BPW = B // NW
L = 16


@functools.cache
def _make_gather():
    mesh = plsc.VectorSubcoreMesh(
        core_axis_name="c", subcore_axis_name="s", num_cores=_NC)

    @functools.partial(
        pl.kernel,
        mesh=mesh,
        out_type=jax.ShapeDtypeStruct((B, DPK), jnp.int32),
        scratch_types=[
            pltpu.VMEM((BPW,), jnp.int32),
            pltpu.VMEM((BPW, DPK), jnp.int32),
            pltpu.SemaphoreType.DMA,
        ],
    )
    def _gather(idx_hbm, table_hbm, out_hbm, idx_v, rows_v, sem):
        wid = lax.axis_index("s") * _NC + lax.axis_index("c")
        base = wid * BPW
        pltpu.sync_copy(idx_hbm.at[pl.ds(base, BPW)], idx_v)

        def group_body(g, _):
            j0 = g * L
            grp = idx_v[pl.ds(j0, L)]
            for k in range(L):
                pltpu.async_copy(
                    table_hbm.at[grp[k]], rows_v.at[j0 + k], sem)

            def drain(k, _):
                pltpu.make_async_copy(
                    table_hbm.at[0], rows_v.at[0], sem).wait()
                return 0

            lax.fori_loop(0, L, drain, 0)
            return 0

        lax.fori_loop(0, BPW // L, group_body, 0)
        pltpu.sync_copy(rows_v, out_hbm.at[pl.ds(base, BPW)])

    return _gather


def _pack_table(t):
    t4 = t.reshape(NPK, PACK, D)
    u = lax.bitcast_convert_type(t4.astype(jnp.bfloat16), jnp.uint16)
    lo = u[:, :, 0::2].astype(jnp.uint32)
    hi = u[:, :, 1::2].astype(jnp.uint32)
    pk = lo | (hi << 16)
    return lax.bitcast_convert_type(pk, jnp.int32).reshape(NPK, DPK)


BB = 512


def _unpack(g, q):
    w = g[:, 0:32]
    for qq in range(1, PACK):
        w = jnp.where(q == qq, g[:, qq * 32:(qq + 1) * 32], w)
    lo = lax.bitcast_convert_type(w.astype(jnp.uint16), jnp.bfloat16)
    hi = lax.bitcast_convert_type(
        (w >> 16).astype(jnp.uint16), jnp.bfloat16)
    return jnp.concatenate(
        [lo.astype(jnp.float32), hi.astype(jnp.float32)], axis=1)


def _mlp_body(poster_ref, w1_ref, b1_ref, w2_ref, b2_ref, wo_ref, bo_ref,
              gu_ref, gi_ref, gv_ref, uq_ref, iq_ref, out_ref):
    h = jnp.dot(poster_ref[...], w1_ref[...],
                preferred_element_type=jnp.float32) + b1_ref[...]
    h = jnp.maximum(h, 0.0)
    iv = jnp.dot(h, w2_ref[...], preferred_element_type=jnp.float32) + b2_ref[...]
    uq = uq_ref[...]
    iq = iq_ref[...]
    umf = _unpack(gu_ref[...], uq)
    imf = _unpack(gi_ref[...], iq)
    uv = _unpack(gv_ref[...], uq)
    vec = umf * imf + uv * iv
    logits = jnp.sum(vec * wo_ref[...], axis=1, keepdims=True) + bo_ref[...]
    out_ref[...] = jax.nn.sigmoid(logits)


def _mlp_combine(poster, w1, b1, w2p, b2p, wop_t, bo, gu, gi, gv, uq, iq):
    return pl.pallas_call(
        _mlp_body,
        grid=(B // BB,),
        in_specs=[
            pl.BlockSpec((BB, POSTER_DIM), lambda i: (i, 0)),
            pl.BlockSpec((POSTER_DIM, HID), lambda i: (0, 0)),
            pl.BlockSpec((1, HID), lambda i: (0, 0)),
            pl.BlockSpec((HID, D), lambda i: (0, 0)),
            pl.BlockSpec((1, D), lambda i: (0, 0)),
            pl.BlockSpec((1, D), lambda i: (0, 0)),
            pl.BlockSpec((1, 1), lambda i: (0, 0)),
            pl.BlockSpec((BB, DPK), lambda i: (i, 0)),
            pl.BlockSpec((BB, DPK), lambda i: (i, 0)),
            pl.BlockSpec((BB, DPK), lambda i: (i, 0)),
            pl.BlockSpec((BB, 1), lambda i: (i, 0)),
            pl.BlockSpec((BB, 1), lambda i: (i, 0)),
        ],
        out_specs=pl.BlockSpec((BB, 1), lambda i: (i, 0)),
        out_shape=jax.ShapeDtypeStruct((B, 1), jnp.float32),
    )(poster, w1, b1, w2p, b2p, wop_t, bo, gu, gi, gv, uq, iq)


def kernel(user_indices, item_indices, poster_embeddings, U_mf, I_mf, U_v,
           W1, b1, W2, b2, Wo, bo):
    ui = user_indices.astype(jnp.int32)
    ii = item_indices.astype(jnp.int32)
    up = ui >> 2
    ip = ii >> 2
    uq = (ui & 3)[:, None]
    iq = (ii & 3)[:, None]

    umf_pk = _pack_table(U_mf)
    imf_pk = _pack_table(I_mf)
    uv_pk = _pack_table(U_v)

    gather = _make_gather()
    gu = gather(up, umf_pk)
    gi = gather(ip, imf_pk)
    gv = gather(up, uv_pk)

    perm = jnp.concatenate([jnp.arange(0, D, 2), jnp.arange(1, D, 2)])
    w2p = W2[:, perm]
    b2p = b2[perm]
    wop = Wo[perm, 0]

    rating = _mlp_combine(
        poster_embeddings, W1,
        b1.reshape(1, HID), w2p, b2p.reshape(1, D),
        wop.reshape(1, D), bo.reshape(1, 1),
        gu, gi, gv, uq, iq)
    return rating


# speedup vs baseline: 7.6444x; 7.6444x over previous
import functools

import jax
import jax.numpy as jnp
from jax import lax
from jax.experimental import pallas as pl
from jax.experimental.pallas import tpu as pltpu
from jax.experimental.pallas import tpu_sc as plsc

B = 4096
D = 64
HID = 128
POSTER_DIM = 2048
NROW = 100000
PACK = 4
DPK = 128
NPK = NROW // PACK

_NC, _NS = 2, 16
NW = _NC * _NS
BPW = B // NW
L = 16


@functools.cache
def _make_gather():
    mesh = plsc.VectorSubcoreMesh(
        core_axis_name="c", subcore_axis_name="s", num_cores=_NC)

    @functools.partial(
        pl.kernel,
        mesh=mesh,
        out_type=jax.ShapeDtypeStruct((B, DPK), jnp.int32),
        scratch_types=[
            pltpu.VMEM((BPW,), jnp.int32),
            pltpu.VMEM((BPW, DPK), jnp.int32),
            pltpu.SemaphoreType.DMA,
        ],
    )
    def _gather(idx_hbm, table_hbm, out_hbm, idx_v, rows_v, sem):
        wid = lax.axis_index("s") * _NC + lax.axis_index("c")
        base = wid * BPW
        pltpu.sync_copy(idx_hbm.at[pl.ds(base, BPW)], idx_v)

        def group_body(g, _):
            j0 = g * L
            grp = idx_v[pl.ds(j0, L)]
            for k in range(L):
                pltpu.async_copy(
                    table_hbm.at[grp[k]], rows_v.at[j0 + k], sem)

            def drain(k, _):
                pltpu.make_async_copy(
                    table_hbm.at[0], rows_v.at[0], sem).wait()
                return 0

            lax.fori_loop(0, L, drain, 0)
            return 0

        lax.fori_loop(0, BPW // L, group_body, 0)
        pltpu.sync_copy(rows_v, out_hbm.at[pl.ds(base, BPW)])

    return _gather


def _pack_table(t):
    t4 = t.reshape(NPK, PACK, D)
    u = lax.bitcast_convert_type(t4.astype(jnp.bfloat16), jnp.uint16)
    lo = u[:, :, 0:32].astype(jnp.uint32)
    hi = u[:, :, 32:64].astype(jnp.uint32)
    pk = lo | (hi << 16)
    return lax.bitcast_convert_type(pk, jnp.int32).reshape(NPK, DPK)


BB = 512


def _unpack(g, q):
    w = g[:, 0:32]
    for qq in range(1, PACK):
        w = jnp.where(q == qq, g[:, qq * 32:(qq + 1) * 32], w)
    lo = lax.bitcast_convert_type(w.astype(jnp.uint16), jnp.bfloat16)
    hi = lax.bitcast_convert_type(
        (w >> 16).astype(jnp.uint16), jnp.bfloat16)
    return jnp.concatenate(
        [lo.astype(jnp.float32), hi.astype(jnp.float32)], axis=1)


def _mlp_body(poster_ref, w1_ref, b1_ref, w2_ref, b2_ref, wo_ref, bo_ref,
              gu_ref, gi_ref, gv_ref, uq_ref, iq_ref, out_ref):
    h = jnp.dot(poster_ref[...], w1_ref[...],
                preferred_element_type=jnp.float32) + b1_ref[...]
    h = jnp.maximum(h, 0.0)
    iv = jnp.dot(h, w2_ref[...], preferred_element_type=jnp.float32) + b2_ref[...]
    uq = uq_ref[...]
    iq = iq_ref[...]
    umf = _unpack(gu_ref[...], uq)
    imf = _unpack(gi_ref[...], iq)
    uv = _unpack(gv_ref[...], uq)
    vec = umf * imf + uv * iv
    logits = jnp.sum(vec * wo_ref[...], axis=1, keepdims=True) + bo_ref[...]
    out_ref[...] = jax.nn.sigmoid(logits)


def _mlp_combine(poster, w1, b1, w2p, b2p, wop_t, bo, gu, gi, gv, uq, iq):
    return pl.pallas_call(
        _mlp_body,
        grid=(B // BB,),
        in_specs=[
            pl.BlockSpec((BB, POSTER_DIM), lambda i: (i, 0)),
            pl.BlockSpec((POSTER_DIM, HID), lambda i: (0, 0)),
            pl.BlockSpec((1, HID), lambda i: (0, 0)),
            pl.BlockSpec((HID, D), lambda i: (0, 0)),
            pl.BlockSpec((1, D), lambda i: (0, 0)),
            pl.BlockSpec((1, D), lambda i: (0, 0)),
            pl.BlockSpec((1, 1), lambda i: (0, 0)),
            pl.BlockSpec((BB, DPK), lambda i: (i, 0)),
            pl.BlockSpec((BB, DPK), lambda i: (i, 0)),
            pl.BlockSpec((BB, DPK), lambda i: (i, 0)),
            pl.BlockSpec((BB, 1), lambda i: (i, 0)),
            pl.BlockSpec((BB, 1), lambda i: (i, 0)),
        ],
        out_specs=pl.BlockSpec((BB, 1), lambda i: (i, 0)),
        out_shape=jax.ShapeDtypeStruct((B, 1), jnp.float32),
    )(poster, w1, b1, w2p, b2p, wop_t, bo, gu, gi, gv, uq, iq)


def kernel(user_indices, item_indices, poster_embeddings, U_mf, I_mf, U_v,
           W1, b1, W2, b2, Wo, bo):
    ui = user_indices.astype(jnp.int32)
    ii = item_indices.astype(jnp.int32)
    up = ui >> 2
    ip = ii >> 2
    uq = (ui & 3)[:, None]
    iq = (ii & 3)[:, None]

    umf_pk = _pack_table(U_mf)
    imf_pk = _pack_table(I_mf)
    uv_pk = _pack_table(U_v)

    gather = _make_gather()
    gu = gather(up, umf_pk)
    gi = gather(ip, imf_pk)
    gv = gather(up, uv_pk)

    rating = _mlp_combine(
        poster_embeddings, W1,
        b1.reshape(1, HID), W2, b2.reshape(1, D),
        Wo.reshape(1, D), bo.reshape(1, 1),
        gu, gi, gv, uq, iq)
    return rating


# speedup vs baseline: 25.8678x; 3.3839x over previous
import functools

import jax
import jax.numpy as jnp
from jax import lax
from jax.experimental import pallas as pl
from jax.experimental.pallas import tpu as pltpu
from jax.experimental.pallas import tpu_sc as plsc

B = 4096
D = 64
HID = 128
POSTER_DIM = 2048

_NC, _NS = 2, 16
NW = _NC * _NS
BPW = B // NW
L = 16


@functools.cache
def _make_gather():
    mesh = plsc.VectorSubcoreMesh(
        core_axis_name="c", subcore_axis_name="s", num_cores=_NC)

    @functools.partial(
        pl.kernel,
        mesh=mesh,
        out_type=jax.ShapeDtypeStruct((B, D), jnp.float32),
        scratch_types=[
            pltpu.VMEM((BPW,), jnp.int32),
            pltpu.VMEM((BPW, D), jnp.float32),
            pltpu.SemaphoreType.DMA,
        ],
    )
    def _gather(idx_hbm, table_hbm, out_hbm, idx_v, rows_v, sem):
        wid = lax.axis_index("s") * _NC + lax.axis_index("c")
        base = wid * BPW
        pltpu.sync_copy(idx_hbm.at[pl.ds(base, BPW)], idx_v)

        def group_body(g, _):
            j0 = g * L
            grp = idx_v[pl.ds(j0, L)]
            for k in range(L):
                pltpu.async_copy(
                    table_hbm.at[grp[k]], rows_v.at[j0 + k], sem)

            def drain(k, _):
                pltpu.make_async_copy(
                    table_hbm.at[0], rows_v.at[0], sem).wait()
                return 0

            lax.fori_loop(0, L, drain, 0)
            return 0

        lax.fori_loop(0, BPW // L, group_body, 0)
        pltpu.sync_copy(rows_v, out_hbm.at[pl.ds(base, BPW)])

    return _gather


BB = 512


def _mlp_body(poster_ref, w1_ref, b1_ref, w2_ref, b2_ref, wo_ref, bo_ref,
              umf_ref, imf_ref, uv_ref, out_ref):
    h = jnp.dot(poster_ref[...], w1_ref[...],
                preferred_element_type=jnp.float32) + b1_ref[...]
    h = jnp.maximum(h, 0.0)
    iv = jnp.dot(h, w2_ref[...], preferred_element_type=jnp.float32) + b2_ref[...]
    vec = umf_ref[...] * imf_ref[...] + uv_ref[...] * iv
    logits = jnp.sum(vec * wo_ref[...], axis=1, keepdims=True) + bo_ref[...]
    out_ref[...] = jax.nn.sigmoid(logits)


def _mlp_combine(poster, w1, b1, w2, b2, wo_t, bo, umf, imf, uv):
    return pl.pallas_call(
        _mlp_body,
        grid=(B // BB,),
        in_specs=[
            pl.BlockSpec((BB, POSTER_DIM), lambda i: (i, 0)),
            pl.BlockSpec((POSTER_DIM, HID), lambda i: (0, 0)),
            pl.BlockSpec((1, HID), lambda i: (0, 0)),
            pl.BlockSpec((HID, D), lambda i: (0, 0)),
            pl.BlockSpec((1, D), lambda i: (0, 0)),
            pl.BlockSpec((1, D), lambda i: (0, 0)),
            pl.BlockSpec((1, 1), lambda i: (0, 0)),
            pl.BlockSpec((BB, D), lambda i: (i, 0)),
            pl.BlockSpec((BB, D), lambda i: (i, 0)),
            pl.BlockSpec((BB, D), lambda i: (i, 0)),
        ],
        out_specs=pl.BlockSpec((BB, 1), lambda i: (i, 0)),
        out_shape=jax.ShapeDtypeStruct((B, 1), jnp.float32),
    )(poster, w1, b1, w2, b2, wo_t, bo, umf, imf, uv)


def kernel(user_indices, item_indices, poster_embeddings, U_mf, I_mf, U_v,
           W1, b1, W2, b2, Wo, bo):
    ui = user_indices.astype(jnp.int32)
    ii = item_indices.astype(jnp.int32)
    gather = _make_gather()
    umf = gather(ui, U_mf)
    imf = gather(ii, I_mf)
    uv = gather(ui, U_v)
    rating = _mlp_combine(
        poster_embeddings, W1,
        b1.reshape(1, HID), W2, b2.reshape(1, D),
        Wo.reshape(1, D), bo.reshape(1, 1),
        umf, imf, uv)
    return rating
